# baseline (speedup 1.0000x reference)
import jax
import jax.numpy as jnp
from jax.experimental import pallas as pl
from jax.experimental.pallas import tpu as pltpu

_C = 16
_N = 1048576
_BLK = 65536


def _fold(x):
    parts = [x[:, 128 * l:128 * (l + 1)] for l in range(_BLK // 128)]
    while len(parts) > 1:
        nxt = [parts[k] + parts[k + 1] for k in range(0, len(parts) - 1, 2)]
        if len(parts) % 2:
            nxt.append(parts[-1])
        parts = nxt
    return parts[0]


def _body(loss_ref, succ_ref, out_ref, at_ref, aa_ref):
    i = pl.program_id(0)

    @pl.when(i == 0)
    def _init():
        at_ref[...] = jnp.zeros_like(at_ref)
        aa_ref[...] = jnp.zeros_like(aa_ref)

    x = loss_ref[...]
    masked = jnp.where(succ_ref[...] == 1, x, 0.0)
    at_ref[...] += _fold(masked)
    aa_ref[...] += _fold(x)

    @pl.when(i == pl.num_programs(0) - 1)
    def _fini():
        ts = jnp.sum(at_ref[...], axis=1, keepdims=True)
        tt = jnp.sum(aa_ref[...], axis=1, keepdims=True)
        lv = jnp.log(ts / tt)
        kl = jnp.maximum(lv * lv - 0.01, 0.0)
        out_ref[...] = jnp.sum(kl, axis=0, keepdims=True)


def kernel(lossTensor, lcSuccesses):
    grid = _N // _BLK
    out = pl.pallas_call(
        _body,
        grid=(grid,),
        in_specs=[
            pl.BlockSpec((_C, _BLK), lambda i: (0, i)),
            pl.BlockSpec((_C, _BLK), lambda i: (0, i)),
        ],
        out_specs=pl.BlockSpec((1, 1), lambda i: (0, 0)),
        out_shape=jax.ShapeDtypeStruct((1, 1), jnp.float32),
        scratch_shapes=[
            pltpu.VMEM((_C, 128), jnp.float32),
            pltpu.VMEM((_C, 128), jnp.float32),
        ],
        compiler_params=pltpu.CompilerParams(
            dimension_semantics=("arbitrary",),
        ),
    )(lossTensor, lcSuccesses)
    return out[0, 0]


# speedup vs baseline: 1.0189x; 1.0189x over previous
import jax
import jax.numpy as jnp
from jax.experimental import pallas as pl
from jax.experimental.pallas import tpu as pltpu

_C = 16
_N = 1048576
_BLK = 65536


def _fold(x):
    acc = x[:, 0:128]
    for l in range(1, _BLK // 128):
        acc = acc + x[:, 128 * l:128 * (l + 1)]
    return acc


def _body(loss_ref, succ_ref, out_ref, at_ref, aa_ref):
    i = pl.program_id(0)

    @pl.when(i == 0)
    def _init():
        at_ref[...] = jnp.zeros_like(at_ref)
        aa_ref[...] = jnp.zeros_like(aa_ref)

    x = loss_ref[...]
    masked = jnp.where(succ_ref[...] == 1, x, 0.0)
    at_ref[...] += _fold(masked)
    aa_ref[...] += _fold(x)

    @pl.when(i == pl.num_programs(0) - 1)
    def _fini():
        ts = jnp.sum(at_ref[...], axis=1, keepdims=True)
        tt = jnp.sum(aa_ref[...], axis=1, keepdims=True)
        lv = jnp.log(ts / tt)
        kl = jnp.maximum(lv * lv - 0.01, 0.0)
        out_ref[...] = jnp.sum(kl, axis=0, keepdims=True)


def kernel(lossTensor, lcSuccesses):
    grid = _N // _BLK
    out = pl.pallas_call(
        _body,
        grid=(grid,),
        in_specs=[
            pl.BlockSpec((_C, _BLK), lambda i: (0, i)),
            pl.BlockSpec((_C, _BLK), lambda i: (0, i)),
        ],
        out_specs=pl.BlockSpec((1, 1), lambda i: (0, 0)),
        out_shape=jax.ShapeDtypeStruct((1, 1), jnp.float32),
        scratch_shapes=[
            pltpu.VMEM((_C, 128), jnp.float32),
            pltpu.VMEM((_C, 128), jnp.float32),
        ],
        compiler_params=pltpu.CompilerParams(
            dimension_semantics=("arbitrary",),
        ),
    )(lossTensor, lcSuccesses)
    return out[0, 0]
